# baseline (speedup 1.0000x reference)
import jax
import jax.numpy as jnp
from jax.experimental import pallas as pl
from jax.experimental.pallas import tpu as pltpu

B, N, F = 4, 2048, 64
NUM_HIDDEN = 5
TR = 512
CH = 512
NCH = N // CH


def _soft(s, a):
    return jnp.where(s > a, s - a, jnp.where(s < -a, s + a, jnp.zeros_like(s)))


def _mm_body(m_ref, v_ref, o_ref):
    o_ref[0] = jnp.dot(m_ref[0], v_ref[0], preferred_element_type=jnp.float32)


@jax.jit
def _mm(m, v):
    return pl.pallas_call(
        _mm_body,
        grid=(B, N // TR),
        in_specs=[
            pl.BlockSpec((1, TR, N), lambda b, t: (b, t, 0)),
            pl.BlockSpec((1, N, F), lambda b, t: (b, 0, 0)),
        ],
        out_specs=pl.BlockSpec((1, TR, F), lambda b, t: (b, t, 0)),
        out_shape=jax.ShapeDtypeStruct((B, N, F), jnp.float32),
    )(m, v)


def _recur_body(a_hbm, d_hbm, e_hbm, bx_ref, al_ref, s_out,
                abuf, dbuf, ebuf, stage, sem):
    b = pl.program_id(0)

    tasks = []
    for src, dst in ((d_hbm, dbuf), (a_hbm, abuf), (e_hbm, ebuf)):
        for k in range(NCH):
            tasks.append((src, dst, k))

    def copy(i):
        src, _, k = tasks[i]
        return pltpu.make_async_copy(
            src.at[b, pl.ds(k * CH, CH), :], stage.at[i % 2], sem.at[i % 2])

    copy(0).start()
    for i in range(len(tasks)):
        if i + 1 < len(tasks):
            copy(i + 1).start()
        copy(i).wait()
        _, dst, k = tasks[i]
        dst[pl.ds(k * CH, CH), :] = stage[i % 2].astype(jnp.bfloat16)

    a = al_ref[0]
    bx = bx_ref[0]
    s = jnp.dot(dbuf[...], bx.astype(jnp.bfloat16),
                preferred_element_type=jnp.float32)
    z = _soft(s, a)
    for _ in range(NUM_HIDDEN - 1):
        y = jnp.dot(abuf[...], s.astype(jnp.bfloat16),
                    preferred_element_type=jnp.float32) + bx
        s = (jnp.dot(dbuf[...], y.astype(jnp.bfloat16),
                     preferred_element_type=jnp.float32) +
             jnp.dot(ebuf[...], z.astype(jnp.bfloat16),
                     preferred_element_type=jnp.float32))
        z = _soft(s, a)
    s_out[0] = s


@jax.jit
def _recurrence(conv_A, conv_D, conv_E, bx, alpha):
    return pl.pallas_call(
        _recur_body,
        grid=(B,),
        in_specs=[
            pl.BlockSpec(memory_space=pl.ANY),
            pl.BlockSpec(memory_space=pl.ANY),
            pl.BlockSpec(memory_space=pl.ANY),
            pl.BlockSpec((1, N, F), lambda b: (b, 0, 0)),
            pl.BlockSpec(memory_space=pltpu.SMEM),
        ],
        out_specs=pl.BlockSpec((1, N, F), lambda b: (b, 0, 0)),
        out_shape=jax.ShapeDtypeStruct((B, N, F), jnp.float32),
        scratch_shapes=[
            pltpu.VMEM((N, N), jnp.bfloat16),
            pltpu.VMEM((N, N), jnp.bfloat16),
            pltpu.VMEM((N, N), jnp.bfloat16),
            pltpu.VMEM((2, CH, N), jnp.float32),
            pltpu.SemaphoreType.DMA((2,)),
        ],
    )(conv_A, conv_D, conv_E, bx, alpha)


@jax.jit
def kernel(x_c, conv_A, conv_B, conv_D, conv_E, conv_H, alpha):
    bx = _mm(conv_B, x_c)
    s = _recurrence(conv_A, conv_D, conv_E, bx, alpha)
    return _mm(conv_H, s)


# speedup vs baseline: 1.0883x; 1.0883x over previous
import jax
import jax.numpy as jnp
from jax.experimental import pallas as pl
from jax.experimental.pallas import tpu as pltpu

B, N, F = 4, 2048, 64
NUM_HIDDEN = 5
TR = 512
CH = 512
NCH = N // CH


def _soft(s, a):
    return jnp.where(s > a, s - a, jnp.where(s < -a, s + a, jnp.zeros_like(s)))


def _mm_body(m_ref, v_ref, o_ref):
    o_ref[0] = jnp.dot(m_ref[0], v_ref[0], preferred_element_type=jnp.float32)


@jax.jit
def _mm(m, v):
    return pl.pallas_call(
        _mm_body,
        grid=(B, N // TR),
        in_specs=[
            pl.BlockSpec((1, TR, N), lambda b, t: (b, t, 0)),
            pl.BlockSpec((1, N, F), lambda b, t: (b, 0, 0)),
        ],
        out_specs=pl.BlockSpec((1, TR, F), lambda b, t: (b, t, 0)),
        out_shape=jax.ShapeDtypeStruct((B, N, F), jnp.float32),
    )(m, v)


def _recur_body(a_hbm, d_hbm, e_hbm, bx_ref, al_ref, s_out,
                abuf, dbuf, ebuf, sem):
    b = pl.program_id(0)

    c_d = pltpu.make_async_copy(d_hbm.at[b], dbuf, sem.at[0])
    c_a = pltpu.make_async_copy(a_hbm.at[b], abuf, sem.at[1])
    c_e = pltpu.make_async_copy(e_hbm.at[b], ebuf, sem.at[2])
    c_d.start()
    c_a.start()
    c_e.start()

    a = al_ref[0]
    bx = bx_ref[0]
    c_d.wait()
    s = jnp.dot(dbuf[...], bx, preferred_element_type=jnp.float32)
    z = _soft(s, a)
    c_a.wait()
    y = jnp.dot(abuf[...], s, preferred_element_type=jnp.float32) + bx
    c_e.wait()
    for it in range(NUM_HIDDEN - 1):
        if it > 0:
            y = jnp.dot(abuf[...], s, preferred_element_type=jnp.float32) + bx
        s = (jnp.dot(dbuf[...], y, preferred_element_type=jnp.float32) +
             jnp.dot(ebuf[...], z, preferred_element_type=jnp.float32))
        z = _soft(s, a)
    s_out[0] = s


@jax.jit
def _recurrence(conv_A, conv_D, conv_E, bx, alpha):
    return pl.pallas_call(
        _recur_body,
        grid=(B,),
        in_specs=[
            pl.BlockSpec(memory_space=pl.ANY),
            pl.BlockSpec(memory_space=pl.ANY),
            pl.BlockSpec(memory_space=pl.ANY),
            pl.BlockSpec((1, N, F), lambda b: (b, 0, 0)),
            pl.BlockSpec(memory_space=pltpu.SMEM),
        ],
        out_specs=pl.BlockSpec((1, N, F), lambda b: (b, 0, 0)),
        out_shape=jax.ShapeDtypeStruct((B, N, F), jnp.float32),
        scratch_shapes=[
            pltpu.VMEM((N, N), jnp.float32),
            pltpu.VMEM((N, N), jnp.float32),
            pltpu.VMEM((N, N), jnp.float32),
            pltpu.SemaphoreType.DMA((3,)),
        ],
    )(conv_A, conv_D, conv_E, bx, alpha)


@jax.jit
def kernel(x_c, conv_A, conv_B, conv_D, conv_E, conv_H, alpha):
    bx = _mm(conv_B, x_c)
    s = _recurrence(conv_A, conv_D, conv_E, bx, alpha)
    return _mm(conv_H, s)
